# baseline (speedup 1.0000x reference)
import jax
import jax.numpy as jnp
from jax.experimental import pallas as pl
from jax.experimental.pallas import tpu as pltpu

B, H, W = 8, 128, 128
C, K, D = 19, 1, 512
Q = B * H * W
BQ = 2048


def _fused_body(emb_ref, logits_ref, means_ref, npw_ref,
                nm_ref, score_ref, py_ref, ps_ref):
    e = emb_ref[...]
    m = means_ref[...]
    lg = logits_ref[...]
    w = npw_ref[...]

    cross = jax.lax.dot_general(
        e, m, (((1,), (1,)), ((), ())),
        preferred_element_type=jnp.float32)
    q2 = jnp.sum(e * e, axis=1, keepdims=True)
    m2 = jnp.sum(m * m, axis=1)[None, :]
    nm = q2 + m2 - 2.0 * cross
    sim = 1.0 / (1.0 + 0.5 * nm)
    score = jax.nn.sigmoid(w[0:1, :] * lg + w[1:2, :] * sim + w[2:3, :])

    mx = jnp.max(lg, axis=1, keepdims=True)
    iota = jax.lax.broadcasted_iota(jnp.int32, lg.shape, 1)
    pred_y = jnp.min(jnp.where(lg == mx, iota, C), axis=1)
    onehot = iota == pred_y[:, None]
    pred_score = jnp.sum(jnp.where(onehot, score, 0.0), axis=1,
                         keepdims=True)

    nm_ref[...] = nm
    score_ref[...] = score
    py_ref[...] = pred_y.astype(jnp.float32)[:, None]
    ps_ref[...] = pred_score


def kernel(emb, logits, means, np_w):
    means2d = means.reshape(C, D)
    npw_t = np_w.T

    grid = (Q // BQ,)
    nm, score, py, ps = pl.pallas_call(
        _fused_body,
        grid=grid,
        in_specs=[
            pl.BlockSpec((BQ, D), lambda i: (i, 0)),
            pl.BlockSpec((BQ, C), lambda i: (i, 0)),
            pl.BlockSpec((C, D), lambda i: (0, 0)),
            pl.BlockSpec((3, C), lambda i: (0, 0)),
        ],
        out_specs=[
            pl.BlockSpec((BQ, C), lambda i: (i, 0)),
            pl.BlockSpec((BQ, C), lambda i: (i, 0)),
            pl.BlockSpec((BQ, 1), lambda i: (i, 0)),
            pl.BlockSpec((BQ, 1), lambda i: (i, 0)),
        ],
        out_shape=[
            jax.ShapeDtypeStruct((Q, C), jnp.float32),
            jax.ShapeDtypeStruct((Q, C), jnp.float32),
            jax.ShapeDtypeStruct((Q, 1), jnp.float32),
            jax.ShapeDtypeStruct((Q, 1), jnp.float32),
        ],
        compiler_params=pltpu.CompilerParams(
            dimension_semantics=("parallel",)),
    )(emb, logits, means2d, npw_t)

    pred_y_f = py.reshape(B, H, W)
    pred_score_r = ps.reshape(B, H, W)
    pred_score_all = score.reshape(B, H, W, C)
    nm_dist_r = nm.reshape(B, H, W, C)
    logits_r = logits.reshape(B, H, W, C)
    return (pred_y_f, pred_score_r, pred_score_all, nm_dist_r, logits_r)


# speedup vs baseline: 1.2257x; 1.2257x over previous
import jax
import jax.numpy as jnp
from jax.experimental import pallas as pl
from jax.experimental.pallas import tpu as pltpu

B, H, W = 8, 128, 128
C, K, D = 19, 1, 512
Q = B * H * W
BQ = 2048


def _fused_body(emb_ref, logits_ref, means_ref, npw_ref,
                nm_ref, score_ref, py_ref, ps_ref):
    e = emb_ref[...]
    m = means_ref[...]
    lg = logits_ref[...]
    w = npw_ref[...]
    lgT = lg.T

    crossT = jax.lax.dot_general(
        m, e, (((1,), (1,)), ((), ())),
        preferred_element_type=jnp.float32)
    ee = e * e
    q2T = jax.lax.dot_general(
        jnp.ones((1, D), jnp.float32), ee, (((1,), (1,)), ((), ())),
        preferred_element_type=jnp.float32)
    m2 = jnp.sum(m * m, axis=1, keepdims=True)

    nmT = q2T + m2 - 2.0 * crossT
    simT = 1.0 / (1.0 + 0.5 * nmT)
    w0 = w[0:1, :].T
    w1 = w[1:2, :].T
    w2 = w[2:3, :].T
    scoreT = jax.nn.sigmoid(w0 * lgT + w1 * simT + w2)

    mxT = jnp.max(lgT, axis=0, keepdims=True)
    iotaT = jax.lax.broadcasted_iota(jnp.int32, lgT.shape, 0)
    pyT = jnp.min(jnp.where(lgT == mxT, iotaT, C), axis=0,
                  keepdims=True)
    onehotT = iotaT == pyT
    psT = jnp.sum(jnp.where(onehotT, scoreT, 0.0), axis=0,
                  keepdims=True)

    nm_ref[...] = nmT.T
    score_ref[...] = scoreT.T
    py_ref[...] = pyT.astype(jnp.float32)
    ps_ref[...] = psT


def kernel(emb, logits, means, np_w):
    means2d = means.reshape(C, D)
    npw_t = np_w.T

    grid = (Q // BQ,)
    nm, score, py, ps = pl.pallas_call(
        _fused_body,
        grid=grid,
        in_specs=[
            pl.BlockSpec((BQ, D), lambda i: (i, 0)),
            pl.BlockSpec((BQ, C), lambda i: (i, 0)),
            pl.BlockSpec((C, D), lambda i: (0, 0)),
            pl.BlockSpec((3, C), lambda i: (0, 0)),
        ],
        out_specs=[
            pl.BlockSpec((BQ, C), lambda i: (i, 0)),
            pl.BlockSpec((BQ, C), lambda i: (i, 0)),
            pl.BlockSpec((1, BQ), lambda i: (0, i)),
            pl.BlockSpec((1, BQ), lambda i: (0, i)),
        ],
        out_shape=[
            jax.ShapeDtypeStruct((Q, C), jnp.float32),
            jax.ShapeDtypeStruct((Q, C), jnp.float32),
            jax.ShapeDtypeStruct((1, Q), jnp.float32),
            jax.ShapeDtypeStruct((1, Q), jnp.float32),
        ],
        compiler_params=pltpu.CompilerParams(
            dimension_semantics=("parallel",)),
    )(emb, logits, means2d, npw_t)

    pred_y_f = py.reshape(B, H, W)
    pred_score_r = ps.reshape(B, H, W)
    pred_score_all = score.reshape(B, H, W, C)
    nm_dist_r = nm.reshape(B, H, W, C)
    logits_r = logits.reshape(B, H, W, C)
    return (pred_y_f, pred_score_r, pred_score_all, nm_dist_r, logits_r)
